# baseline (speedup 1.0000x reference)
import jax
import jax.numpy as jnp
from jax.experimental import pallas as pl
from jax.experimental.pallas import tpu as pltpu

D_MODEL = 2048
N_EXPERTS = 64
TOP_K = 8
BLOCK_TOKENS = 512


def _router_body(x_hbm, w_ref, out_w_ref, out_i_ref, xbuf, sem):
    n_tokens = x_hbm.shape[0]
    n_blocks = n_tokens // BLOCK_TOKENS
    w = w_ref[...]

    def x_copy(i, slot):
        return pltpu.make_async_copy(
            x_hbm.at[pl.ds(i * BLOCK_TOKENS, BLOCK_TOKENS), :],
            xbuf.at[slot],
            sem.at[slot],
        )

    x_copy(0, 0).start()
    for i in range(n_blocks):
        if i + 1 < n_blocks:
            x_copy(i + 1, (i + 1) % 2).start()
        x_copy(i, i % 2).wait()
        x = xbuf[i % 2]

        logits = jax.lax.dot_general(
            w, x, (((0,), (1,)), ((), ())),
            preferred_element_type=jnp.float32)

        m = jnp.max(logits, axis=0, keepdims=True)
        e = jnp.exp(logits - m)
        s = jnp.sum(e, axis=0, keepdims=True)
        probs = e / s

        lane = jax.lax.broadcasted_iota(
            jnp.int32, probs.shape, 0).astype(jnp.float32)
        vals = []
        idxs = []
        p = probs
        for k in range(TOP_K):
            mk = jnp.max(p, axis=0, keepdims=True)
            ik = jnp.min(jnp.where(p == mk, lane, float(N_EXPERTS)),
                         axis=0, keepdims=True)
            vals.append(mk)
            idxs.append(ik)
            if k + 1 < TOP_K:
                p = jnp.where(lane == ik, -1.0, p)

        top_w = jnp.concatenate(vals, axis=0)
        top_i = jnp.concatenate(idxs, axis=0)
        top_w = top_w / (jnp.sum(top_w, axis=0, keepdims=True) + 1e-9)

        cols = pl.ds(i * BLOCK_TOKENS, BLOCK_TOKENS)
        out_w_ref[:, cols] = top_w
        out_i_ref[:, cols] = top_i.astype(jnp.int32)


def kernel(x, W_t):
    n_tokens = x.shape[0]
    out_w_t, out_i_t = pl.pallas_call(
        _router_body,
        in_specs=[
            pl.BlockSpec(memory_space=pltpu.HBM),
            pl.BlockSpec(memory_space=pltpu.VMEM),
        ],
        out_specs=[
            pl.BlockSpec(memory_space=pltpu.VMEM),
            pl.BlockSpec(memory_space=pltpu.VMEM),
        ],
        out_shape=[
            jax.ShapeDtypeStruct((TOP_K, n_tokens), jnp.float32),
            jax.ShapeDtypeStruct((TOP_K, n_tokens), jnp.int32),
        ],
        scratch_shapes=[
            pltpu.VMEM((2, BLOCK_TOKENS, D_MODEL), jnp.float32),
            pltpu.SemaphoreType.DMA((2,)),
        ],
    )(x, W_t)
    return out_w_t.T, out_i_t.T.astype(jnp.int64)


# speedup vs baseline: 1.0921x; 1.0921x over previous
import jax
import jax.numpy as jnp
from jax.experimental import pallas as pl
from jax.experimental.pallas import tpu as pltpu

D_MODEL = 2048
N_EXPERTS = 64
TOP_K = 8
BLOCK_TOKENS = 1024
N_BUF = 3


def _router_body(x_hbm, w_ref, out_w_ref, out_i_ref, xbuf, sem):
    n_tokens = x_hbm.shape[0]
    n_blocks = n_tokens // BLOCK_TOKENS
    w = w_ref[...]

    def x_copy(i, slot):
        return pltpu.make_async_copy(
            x_hbm.at[pl.ds(i * BLOCK_TOKENS, BLOCK_TOKENS), :],
            xbuf.at[slot],
            sem.at[slot],
        )

    for j in range(N_BUF - 1):
        x_copy(j, j % N_BUF).start()
    for i in range(n_blocks):
        if i + N_BUF - 1 < n_blocks:
            x_copy(i + N_BUF - 1, (i + N_BUF - 1) % N_BUF).start()
        x_copy(i, i % N_BUF).wait()
        x = xbuf[i % N_BUF]

        logits = jax.lax.dot_general(
            w, x, (((0,), (1,)), ((), ())),
            preferred_element_type=jnp.float32)

        m = jnp.max(logits, axis=0, keepdims=True)
        e = jnp.exp(logits - m)
        s = jnp.sum(e, axis=0, keepdims=True)
        probs = e / s

        lane = jax.lax.broadcasted_iota(
            jnp.int32, probs.shape, 0).astype(jnp.float32)
        vals = []
        idxs = []
        p = probs
        for k in range(TOP_K):
            mk = jnp.max(p, axis=0, keepdims=True)
            ik = jnp.min(jnp.where(p == mk, lane, float(N_EXPERTS)),
                         axis=0, keepdims=True)
            vals.append(mk)
            idxs.append(ik)
            if k + 1 < TOP_K:
                p = jnp.where(lane == ik, -1.0, p)

        top_w = jnp.concatenate(vals, axis=0)
        top_i = jnp.concatenate(idxs, axis=0)
        top_w = top_w / (jnp.sum(top_w, axis=0, keepdims=True) + 1e-9)

        cols = pl.ds(i * BLOCK_TOKENS, BLOCK_TOKENS)
        out_w_ref[:, cols] = top_w
        out_i_ref[:, cols] = top_i.astype(jnp.int32)


def kernel(x, W_t):
    n_tokens = x.shape[0]
    out_w_t, out_i_t = pl.pallas_call(
        _router_body,
        in_specs=[
            pl.BlockSpec(memory_space=pltpu.HBM),
            pl.BlockSpec(memory_space=pltpu.VMEM),
        ],
        out_specs=[
            pl.BlockSpec(memory_space=pltpu.VMEM),
            pl.BlockSpec(memory_space=pltpu.VMEM),
        ],
        out_shape=[
            jax.ShapeDtypeStruct((TOP_K, n_tokens), jnp.float32),
            jax.ShapeDtypeStruct((TOP_K, n_tokens), jnp.int32),
        ],
        scratch_shapes=[
            pltpu.VMEM((N_BUF, BLOCK_TOKENS, D_MODEL), jnp.float32),
            pltpu.SemaphoreType.DMA((N_BUF,)),
        ],
    )(x, W_t)
    return out_w_t.T, out_i_t.T.astype(jnp.int64)


# speedup vs baseline: 1.1235x; 1.0288x over previous
import jax
import jax.numpy as jnp
from jax.experimental import pallas as pl
from jax.experimental.pallas import tpu as pltpu

D_MODEL = 2048
N_EXPERTS = 64
TOP_K = 8
BLOCK_TOKENS = 1024
N_BUF = 2


def _router_body(x_hbm, w_ref, out_w_ref, out_i_ref, xbuf, sem):
    n_tokens = x_hbm.shape[0]
    n_blocks = n_tokens // BLOCK_TOKENS
    w = w_ref[...]

    def x_copy(i, slot):
        return pltpu.make_async_copy(
            x_hbm.at[pl.ds(i * BLOCK_TOKENS, BLOCK_TOKENS), :],
            xbuf.at[slot],
            sem.at[slot],
        )

    for j in range(N_BUF - 1):
        x_copy(j, j % N_BUF).start()
    for i in range(n_blocks):
        if i + N_BUF - 1 < n_blocks:
            x_copy(i + N_BUF - 1, (i + N_BUF - 1) % N_BUF).start()
        x_copy(i, i % N_BUF).wait()
        x = xbuf[i % N_BUF]

        logits = jax.lax.dot_general(
            w, x, (((0,), (1,)), ((), ())),
            preferred_element_type=jnp.float32)

        m = jnp.max(logits, axis=0, keepdims=True)
        e = jnp.exp(logits - m)
        s = jnp.sum(e, axis=0, keepdims=True)
        probs = e / s

        lane = jax.lax.broadcasted_iota(
            jnp.int32, probs.shape, 0).astype(jnp.float32)
        vals = []
        idxs = []
        p = probs
        for k in range(TOP_K):
            mk = jnp.max(p, axis=0, keepdims=True)
            ik = jnp.min(jnp.where(p == mk, lane, float(N_EXPERTS)),
                         axis=0, keepdims=True)
            vals.append(mk)
            idxs.append(ik)
            if k + 1 < TOP_K:
                p = jnp.where(lane == ik, -1.0, p)

        top_w = jnp.concatenate(vals, axis=0)
        top_i = jnp.concatenate(idxs, axis=0)
        top_w = top_w / (jnp.sum(top_w, axis=0, keepdims=True) + 1e-9)

        cols = pl.ds(i * BLOCK_TOKENS, BLOCK_TOKENS)
        out_w_ref[:, cols] = top_w
        out_i_ref[:, cols] = top_i.astype(jnp.int32)


def kernel(x, W_t):
    n_tokens = x.shape[0]
    out_w_t, out_i_t = pl.pallas_call(
        _router_body,
        in_specs=[
            pl.BlockSpec(memory_space=pltpu.HBM),
            pl.BlockSpec(memory_space=pltpu.VMEM),
        ],
        out_specs=[
            pl.BlockSpec(memory_space=pltpu.VMEM),
            pl.BlockSpec(memory_space=pltpu.VMEM),
        ],
        out_shape=[
            jax.ShapeDtypeStruct((TOP_K, n_tokens), jnp.float32),
            jax.ShapeDtypeStruct((TOP_K, n_tokens), jnp.int32),
        ],
        scratch_shapes=[
            pltpu.VMEM((N_BUF, BLOCK_TOKENS, D_MODEL), jnp.float32),
            pltpu.SemaphoreType.DMA((N_BUF,)),
        ],
    )(x, W_t)
    return out_w_t.T, out_i_t.T.astype(jnp.int64)
